# baseline (speedup 1.0000x reference)
import functools

import jax
import jax.numpy as jnp
from jax import lax
from jax.experimental import pallas as pl
from jax.experimental.pallas import tpu as pltpu
from jax.experimental.pallas import tpu_sc as plsc

N_NODES = 10000
N_EDGES = 320000
D = 128

NC = 2
NS = 16
NW = NC * NS
LANES = 16

EPW = N_EDGES // NW
C = 80
NCHUNK = EPW // C
N_PAD = 10240
RPS = N_PAD // NS
ZR = 64


def _tanh_body(x_ref, o_ref):
    o_ref[...] = jnp.tanh(x_ref[...])


def _tanh_tc(x):
    return pl.pallas_call(
        _tanh_body,
        out_shape=jax.ShapeDtypeStruct((N_NODES, D), jnp.float32),
        grid=(10,),
        in_specs=[pl.BlockSpec((N_NODES // 10, D), lambda i: (i, 0))],
        out_specs=pl.BlockSpec((N_NODES // 10, D), lambda i: (i, 0)),
    )(x)


def _add_body(p_ref, o_ref):
    o_ref[...] = p_ref[0] + p_ref[1]


def _add_tc(partials):
    return pl.pallas_call(
        _add_body,
        out_shape=jax.ShapeDtypeStruct((N_NODES, D), jnp.float32),
        grid=(10,),
        in_specs=[pl.BlockSpec((NC, N_NODES // 10, D), lambda i: (0, i, 0))],

        out_specs=pl.BlockSpec((N_NODES // 10, D), lambda i: (i, 0)),
    )(partials)


def _sc_scatter(t, ei_flat, w):
    mesh = plsc.VectorSubcoreMesh(
        core_axis_name="c", subcore_axis_name="s", num_cores=NC,
        num_subcores=NS)

    @functools.partial(
        pl.kernel,
        mesh=mesh,
        out_type=jax.ShapeDtypeStruct((NC, N_PAD, D), jnp.float32),
        scratch_types=[
            [pltpu.VMEM((C,), jnp.int32) for _ in range(3)],
            [pltpu.VMEM((C,), jnp.int32) for _ in range(3)],
            [pltpu.VMEM((C,), jnp.float32) for _ in range(3)],
            [pltpu.VMEM((C, D), jnp.float32) for _ in range(3)],
            [pltpu.VMEM((C,), jnp.int32) for _ in range(3)],
            pltpu.VMEM((ZR, D), jnp.float32),
            pltpu.VMEM_SHARED((N_PAD, D), jnp.float32),
            [pltpu.SemaphoreType.DMA for _ in range(3)],
            [pltpu.SemaphoreType.DMA for _ in range(3)],
            pltpu.SemaphoreType.DMA,
        ],
    )
    def k(t_hbm, ei_hbm, w_hbm, out_hbm,
          sbufs, dbufs, wbufs, bufs, srcs, zbuf, acc, gsems, ssems, sem_i):
        cid = lax.axis_index("c")
        sid = lax.axis_index("s")
        wid = sid * NC + cid
        cbase = wid * NCHUNK
        row0 = sid * RPS

        def zrow(i, _):
            for kk in range(D // LANES):
                zbuf[i, pl.ds(kk * LANES, LANES)] = jnp.zeros(
                    (LANES,), jnp.float32)
            return _
        lax.fori_loop(0, ZR, zrow, 0)
        for j in range(RPS // ZR):
            pltpu.sync_copy(zbuf, acc.at[pl.ds(row0 + j * ZR, ZR)])
        plsc.subcore_barrier()

        def idx_start(j, r):
            e0 = (cbase + j) * C
            pltpu.make_async_copy(
                ei_hbm.at[pl.ds(e0, C)], sbufs[r], sem_i).start()
            pltpu.make_async_copy(
                ei_hbm.at[pl.ds(N_EDGES + e0, C)], dbufs[r], sem_i).start()
            pltpu.make_async_copy(
                w_hbm.at[pl.ds(e0, C)], wbufs[r], sem_i).start()

        def idx_wait(r):
            e0 = cbase * C
            pltpu.make_async_copy(
                ei_hbm.at[pl.ds(e0, C)], sbufs[r], sem_i).wait()
            pltpu.make_async_copy(
                ei_hbm.at[pl.ds(N_EDGES + e0, C)], dbufs[r], sem_i).wait()
            pltpu.make_async_copy(
                w_hbm.at[pl.ds(e0, C)], wbufs[r], sem_i).wait()

        def gather(j, r):
            return pltpu.make_async_copy(
                t_hbm.at[dbufs[r]], bufs[r], gsems[r])

        def scatter(r):
            return pltpu.make_async_copy(
                bufs[r], acc.at[srcs[r]], ssems[r])

        def scale_and_scatter(r):
            buf, wbuf = bufs[r], wbufs[r]

            def scale(g, _):
                wvec = wbuf[pl.ds(g * LANES, LANES)]
                ws = [wvec[l] for l in range(LANES)]
                for kk in range(D // LANES):
                    sl = pl.ds(kk * LANES, LANES)
                    for l in range(LANES):
                        e = g * LANES + l
                        buf[e, sl] = buf[e, sl] * ws[l]
                return _
            lax.fori_loop(0, C // LANES, scale, 0)
            for g in range(C // LANES):
                sl = pl.ds(g * LANES, LANES)
                srcs[r][sl] = sbufs[r][sl]
            scatter(r).start(add=True)

        idx_start(0, 0)
        idx_wait(0)
        gather(0, 0).start()
        idx_start(1, 1)

        def step(j, cs, first):
            ns = (cs + 1) % 3
            idx_wait(ns)
            if not first:
                scatter(ns).wait()
            gather(j + 1, ns).start()
            idx_start(j + 2, (cs + 2) % 3)
            gather(j, cs).wait()
            scale_and_scatter(cs)

        step(0, 0, True)
        step(1, 1, True)
        step(2, 2, False)

        def triple(q, _):
            j0 = 3 * q
            step(j0, 0, False)
            step(j0 + 1, 1, False)
            step(j0 + 2, 2, False)
            return _
        lax.fori_loop(1, (NCHUNK - 2) // 3, triple, 0)

        idx_wait(1)
        scatter(1).wait()
        gather(NCHUNK - 1, 1).start()
        gather(NCHUNK - 2, 0).wait()
        scale_and_scatter(0)
        gather(NCHUNK - 1, 1).wait()
        scale_and_scatter(1)
        scatter(2).wait()
        scatter(0).wait()
        scatter(1).wait()
        plsc.subcore_barrier()

        for j in range(RPS // ZR):
            r = row0 + j * ZR
            pltpu.sync_copy(acc.at[pl.ds(r, ZR)], zbuf)
            pltpu.sync_copy(zbuf, out_hbm.at[cid, pl.ds(r, ZR)])

    return k(t, ei_flat, w)


def kernel(x, edge_index, weight):
    ei_flat = edge_index.astype(jnp.int32).reshape(2 * N_EDGES)
    t = _tanh_tc(x)
    partials = _sc_scatter(t, ei_flat, weight)
    return _add_tc(partials)


# speedup vs baseline: 1.0535x; 1.0535x over previous
import functools

import jax
import jax.numpy as jnp
from jax import lax
from jax.experimental import pallas as pl
from jax.experimental.pallas import tpu as pltpu
from jax.experimental.pallas import tpu_sc as plsc

N_NODES = 10000
N_EDGES = 320000
D = 128

NC = 2
NS = 16
NW = NC * NS
LANES = 16

EPW = N_EDGES // NW
C = 80
NCHUNK = EPW // C
N_PAD = 10240
RPS = N_PAD // NS
ZR = 64


def _tanh_body(x_ref, o_ref):
    o_ref[...] = jnp.tanh(x_ref[...])


def _tanh_tc(x):
    return pl.pallas_call(
        _tanh_body,
        out_shape=jax.ShapeDtypeStruct((N_NODES, D), jnp.float32),
        grid=(5,),
        in_specs=[pl.BlockSpec((N_NODES // 5, D), lambda i: (i, 0))],
        out_specs=pl.BlockSpec((N_NODES // 5, D), lambda i: (i, 0)),
    )(x)


def _add_body(p_ref, o_ref):
    o_ref[...] = p_ref[0] + p_ref[1]


def _add_tc(partials):
    return pl.pallas_call(
        _add_body,
        out_shape=jax.ShapeDtypeStruct((N_NODES, D), jnp.float32),
        grid=(5,),
        in_specs=[pl.BlockSpec((NC, N_NODES // 5, D), lambda i: (0, i, 0))],

        out_specs=pl.BlockSpec((N_NODES // 5, D), lambda i: (i, 0)),
    )(partials)


def _sc_scatter(t, ei_flat, w):
    mesh = plsc.VectorSubcoreMesh(
        core_axis_name="c", subcore_axis_name="s", num_cores=NC,
        num_subcores=NS)

    @functools.partial(
        pl.kernel,
        mesh=mesh,
        out_type=jax.ShapeDtypeStruct((NC, N_PAD, D), jnp.float32),
        scratch_types=[
            [pltpu.VMEM((C,), jnp.int32) for _ in range(3)],
            [pltpu.VMEM((C,), jnp.int32) for _ in range(3)],
            [pltpu.VMEM((C,), jnp.float32) for _ in range(3)],
            [pltpu.VMEM((C, D), jnp.float32) for _ in range(3)],
            [pltpu.VMEM((C,), jnp.int32) for _ in range(3)],
            pltpu.VMEM((ZR, D), jnp.float32),
            pltpu.VMEM_SHARED((N_PAD, D), jnp.float32),
            [pltpu.SemaphoreType.DMA for _ in range(3)],
            [pltpu.SemaphoreType.DMA for _ in range(3)],
            pltpu.SemaphoreType.DMA,
        ],
    )
    def k(t_hbm, ei_hbm, w_hbm, out_hbm,
          sbufs, dbufs, wbufs, bufs, srcs, zbuf, acc, gsems, ssems, sem_i):
        cid = lax.axis_index("c")
        sid = lax.axis_index("s")
        wid = sid * NC + cid
        cbase = wid * NCHUNK
        row0 = sid * RPS

        def zrow(i, _):
            for kk in range(D // LANES):
                zbuf[i, pl.ds(kk * LANES, LANES)] = jnp.zeros(
                    (LANES,), jnp.float32)
            return _
        lax.fori_loop(0, ZR, zrow, 0)
        for j in range(RPS // ZR):
            pltpu.sync_copy(zbuf, acc.at[pl.ds(row0 + j * ZR, ZR)])
        plsc.subcore_barrier()

        def idx_start(j, r):
            e0 = (cbase + j) * C
            pltpu.make_async_copy(
                ei_hbm.at[pl.ds(e0, C)], sbufs[r], sem_i).start()
            pltpu.make_async_copy(
                ei_hbm.at[pl.ds(N_EDGES + e0, C)], dbufs[r], sem_i).start()
            pltpu.make_async_copy(
                w_hbm.at[pl.ds(e0, C)], wbufs[r], sem_i).start()

        def idx_wait(r):
            e0 = cbase * C
            pltpu.make_async_copy(
                ei_hbm.at[pl.ds(e0, C)], sbufs[r], sem_i).wait()
            pltpu.make_async_copy(
                ei_hbm.at[pl.ds(N_EDGES + e0, C)], dbufs[r], sem_i).wait()
            pltpu.make_async_copy(
                w_hbm.at[pl.ds(e0, C)], wbufs[r], sem_i).wait()

        def gather(j, r):
            return pltpu.make_async_copy(
                t_hbm.at[dbufs[r]], bufs[r], gsems[r])

        def scatter(r):
            return pltpu.make_async_copy(
                bufs[r], acc.at[srcs[r]], ssems[r])

        def scale_and_scatter(r):
            buf, wbuf = bufs[r], wbufs[r]

            def scale(g, _):
                wvec = wbuf[pl.ds(g * LANES, LANES)]
                for l in range(LANES):
                    ws = wvec[l]
                    e = g * LANES + l
                    for kk in range(D // LANES):
                        sl = pl.ds(kk * LANES, LANES)
                        buf[e, sl] = buf[e, sl] * ws
                return _
            lax.fori_loop(0, C // LANES, scale, 0)
            for g in range(C // LANES):
                sl = pl.ds(g * LANES, LANES)
                srcs[r][sl] = sbufs[r][sl]
            scatter(r).start(add=True)

        idx_start(0, 0)
        idx_wait(0)
        gather(0, 0).start()
        idx_start(1, 1)

        def step(j, cs, first):
            ns = (cs + 1) % 3
            idx_wait(ns)
            if not first:
                scatter(ns).wait()
            gather(j + 1, ns).start()
            idx_start(j + 2, (cs + 2) % 3)
            gather(j, cs).wait()
            scale_and_scatter(cs)

        step(0, 0, True)
        step(1, 1, True)
        step(2, 2, False)

        def triple(q, _):
            j0 = 3 * q
            step(j0, 0, False)
            step(j0 + 1, 1, False)
            step(j0 + 2, 2, False)
            return _
        lax.fori_loop(1, (NCHUNK - 2) // 3, triple, 0)

        idx_wait(1)
        scatter(1).wait()
        gather(NCHUNK - 1, 1).start()
        gather(NCHUNK - 2, 0).wait()
        scale_and_scatter(0)
        gather(NCHUNK - 1, 1).wait()
        scale_and_scatter(1)
        scatter(2).wait()
        scatter(0).wait()
        scatter(1).wait()
        plsc.subcore_barrier()

        for j in range(RPS // ZR):
            r = row0 + j * ZR
            pltpu.sync_copy(acc.at[pl.ds(r, ZR)], zbuf)
            pltpu.sync_copy(zbuf, out_hbm.at[cid, pl.ds(r, ZR)])

    return k(t, ei_flat, w)


def kernel(x, edge_index, weight):
    ei_flat = edge_index.astype(jnp.int32).reshape(2 * N_EDGES)
    t = _tanh_tc(x)
    partials = _sc_scatter(t, ei_flat, weight)
    return _add_tc(partials)
